# baseline (speedup 1.0000x reference)
import jax
import jax.numpy as jnp
from jax import lax
from jax.experimental import pallas as pl
from jax.experimental.pallas import tpu as pltpu
from jax.experimental.pallas import tpu_sc as plsc

N_ATOMS = 10000
N_EDGES = 320000
N_MOLS = 500
NF = 128
NG = 32
N_CONV = 3
CUTOFF = 5.0

NC, NS = 2, 16
NW = NC * NS
EP = 327680
NP = 10240
ROWS_E = EP // 128

A_ROWS = EP // 128 // NW


def _sc_d2_body(x_hbm, y_hbm, z_hbm, a0_hbm, a1_hbm, d2_hbm,
                xv, yv, zv, i0v, i1v, d2v):
    c = lax.axis_index("c")
    s = lax.axis_index("s")
    wid = s * NC + c
    pltpu.sync_copy(x_hbm, xv)
    pltpu.sync_copy(y_hbm, yv)
    pltpu.sync_copy(z_hbm, zv)
    base = wid * A_ROWS
    pltpu.sync_copy(a0_hbm.at[pl.ds(base, A_ROWS)], i0v)
    pltpu.sync_copy(a1_hbm.at[pl.ds(base, A_ROWS)], i1v)

    def body(i, _):
        r = i // 8
        co = (i % 8) * 16
        idx0 = i0v[r, pl.ds(co, 16)]
        idx1 = i1v[r, pl.ds(co, 16)]
        dx = plsc.load_gather(xv, [idx1]) - plsc.load_gather(xv, [idx0])
        dy = plsc.load_gather(yv, [idx1]) - plsc.load_gather(yv, [idx0])
        dz = plsc.load_gather(zv, [idx1]) - plsc.load_gather(zv, [idx0])
        d2v[r, pl.ds(co, 16)] = dx * dx + dy * dy + dz * dz + 1e-12
        return 0

    lax.fori_loop(0, A_ROWS * 8, body, 0)
    pltpu.sync_copy(d2v, d2_hbm.at[pl.ds(base, A_ROWS)])


def _sc_d2(xp, yp, zp, a0r, a1r):
    mesh = plsc.VectorSubcoreMesh(core_axis_name="c", subcore_axis_name="s")
    f = pl.kernel(
        _sc_d2_body,
        out_type=jax.ShapeDtypeStruct((ROWS_E, 128), jnp.float32),
        mesh=mesh,
        compiler_params=pltpu.CompilerParams(needs_layout_passes=False),
        scratch_types=[
            pltpu.VMEM((NP,), jnp.float32),
            pltpu.VMEM((NP,), jnp.float32),
            pltpu.VMEM((NP,), jnp.float32),
            pltpu.VMEM((A_ROWS, 128), jnp.int32),
            pltpu.VMEM((A_ROWS, 128), jnp.int32),
            pltpu.VMEM((A_ROWS, 128), jnp.float32),
        ],
    )
    return f(xp, yp, zp, a0r, a1r)


KB = 64
EPC = EP // NC
CHUNKS = EPC // NS // KB
NSH = 10112
NST = NSH // NS


RPC = 2 * KB
BCH = 20
CH0 = 180
CH1 = 140


def _sc_agg_body(isrc_hbm, idst_hbm, w_hbm, rn_hbm, z_hbm, agg_hbm, agg_sh,
                 ibs, ibd, ga, gb, wv, sga, sgb, sw, sca, scb):
    c = lax.axis_index("c")
    s = lax.axis_index("s")
    g = (ga, gb)
    sg = (sga, sgb)
    sc = (sca, scb)
    nch = jnp.where(c == 0, CH0, CH1)
    tchunk = jnp.where(c == 0, s * CH0, NS * CH0 + s * CH1)

    def gfire(lc, b):
        pltpu.async_copy(rn_hbm.at[ibs.at[pl.ds(lc * RPC, RPC)]], g[b], sg[b])

    def gdrain(b):
        pltpu.make_async_copy(
            rn_hbm.at[ibs.at[pl.ds(0, RPC)]], g[b], sg[b]).wait()

    def scfire(lc, b):
        pltpu.async_copy(
            g[b], agg_sh.at[ibd.at[pl.ds(lc * RPC, RPC)]], sc[b], add=True)

    def scdrain(b):
        pltpu.make_async_copy(
            g[b], agg_sh.at[ibd.at[pl.ds(0, RPC)]], sc[b]).wait()

    def wfire(gc):
        pltpu.async_copy(w_hbm.at[pl.ds(gc * KB, KB)], wv, sw)

    def wdrain():
        pltpu.make_async_copy(w_hbm.at[pl.ds(0, KB)], wv, sw).wait()

    pltpu.sync_copy(z_hbm, agg_sh.at[pl.ds(s * NST, NST)])
    plsc.subcore_barrier()

    def batch(m, _):
        @pl.when(m >= 1)
        def _():
            scdrain(1)
        base = tchunk + m * BCH
        pltpu.sync_copy(isrc_hbm.at[pl.ds(base * RPC, BCH * RPC)], ibs)
        pltpu.sync_copy(idst_hbm.at[pl.ds(base * RPC, BCH * RPC)], ibd)
        gfire(0, 0)
        wfire(base)

        def pair(i, _):
            for b in range(2):
                j = 2 * i + b
                nb = 1 - b
                if b == 0:
                    @pl.when(i >= 1)
                    def _():
                        scdrain(nb)
                    gfire(j + 1, nb)
                else:
                    scdrain(nb)

                    @pl.when(i < BCH // 2 - 1)
                    def _():
                        gfire(j + 1, nb)
                gdrain(b)
                wdrain()

                def mul(t, _):
                    for f in range(NF // 16):
                        sl = pl.ds(f * 16, 16)
                        wj = wv[t, sl]
                        g[b][t, sl] = g[b][t, sl] * wj
                        g[b][t + KB, sl] = g[b][t + KB, sl] * wj
                    return 0

                lax.fori_loop(0, KB, mul, 0, unroll=4)
                scfire(j, b)
                @pl.when(j < BCH - 1)
                def _():
                    wfire(base + j + 1)
            return 0

        lax.fori_loop(0, BCH // 2, pair, 0)
        return 0

    lax.fori_loop(0, nch // BCH, batch, 0)
    scdrain(1)
    plsc.subcore_barrier()
    pltpu.sync_copy(agg_sh.at[pl.ds(s * NST, NST)],
                    agg_hbm.at[c, pl.ds(s * NST, NST)])


def _sc_agg(isrc, idst, wedge, rn, zrows):
    mesh = plsc.VectorSubcoreMesh(core_axis_name="c", subcore_axis_name="s")
    f = pl.kernel(
        _sc_agg_body,
        out_type=jax.ShapeDtypeStruct((NC, NP, NF), jnp.float32),
        mesh=mesh,
        compiler_params=pltpu.CompilerParams(needs_layout_passes=False),
        scratch_types=[
            pltpu.VMEM_SHARED((NSH, NF), jnp.float32),
            pltpu.VMEM((BCH * RPC,), jnp.int32),
            pltpu.VMEM((BCH * RPC,), jnp.int32),
            pltpu.VMEM((RPC, NF), jnp.float32),
            pltpu.VMEM((RPC, NF), jnp.float32),
            pltpu.VMEM((KB, NF), jnp.float32),
            pltpu.SemaphoreType.DMA,
            pltpu.SemaphoreType.DMA,
            pltpu.SemaphoreType.DMA,
            pltpu.SemaphoreType.DMA,
            pltpu.SemaphoreType.DMA,
        ],
    )
    return f(isrc, idst, wedge, rn, zrows)


def _ssp(x):
    return jnp.logaddexp(x, 0.0) - jnp.log(2.0)


EB = 2048


def _tc_wedge_body(d2_ref, w1_ref, b1_ref, w2_ref, b2_ref, out_ref):
    width = CUTOFF / (NG - 1)
    coeff = -0.5 / (width * width)
    dist = jnp.sqrt(d2_ref[...])
    offs = lax.broadcasted_iota(jnp.int32, (EB, NG), 1).astype(jnp.float32) * width
    eg = jnp.exp(coeff * (dist - offs) ** 2)
    h = _ssp(jnp.dot(eg, w1_ref[...],
                     preferred_element_type=jnp.float32) + b1_ref[...])
    out_ref[...] = jnp.dot(h, w2_ref[...],
                           preferred_element_type=jnp.float32) + b2_ref[...]


def _tc_wedge(d2, w1, b1, w2, b2):
    return pl.pallas_call(
        _tc_wedge_body,
        grid=(EP // EB,),
        in_specs=[
            pl.BlockSpec((EB, 1), lambda e: (e, 0)),
            pl.BlockSpec((NG, NG), lambda e: (0, 0)),
            pl.BlockSpec((1, NG), lambda e: (0, 0)),
            pl.BlockSpec((NG, NF), lambda e: (0, 0)),
            pl.BlockSpec((1, NF), lambda e: (0, 0)),
        ],
        out_specs=pl.BlockSpec((EB, NF), lambda e: (e, 0)),
        out_shape=jax.ShapeDtypeStruct((EP, NF), jnp.float32),
    )(d2, w1, b1, w2, b2)


AB = 2048


def _tc_embed_body(z_ref, emb_ref, wn_ref, bn_ref, r_ref, rn_ref):
    cols = lax.broadcasted_iota(jnp.int32, (AB, NF), 1)
    oh = (z_ref[...] == cols).astype(jnp.float32)
    r = jnp.dot(oh, emb_ref[...], preferred_element_type=jnp.float32)
    rn = jnp.dot(r, wn_ref[...],
                 preferred_element_type=jnp.float32) + bn_ref[...]
    r_ref[...] = r
    rn_ref[...] = rn


def _tc_embed(z2, emb_pad, wn, bn):
    return pl.pallas_call(
        _tc_embed_body,
        grid=(NP // AB,),
        in_specs=[
            pl.BlockSpec((AB, 1), lambda a: (a, 0)),
            pl.BlockSpec((NF, NF), lambda a: (0, 0)),
            pl.BlockSpec((NF, NF), lambda a: (0, 0)),
            pl.BlockSpec((1, NF), lambda a: (0, 0)),
        ],
        out_specs=[
            pl.BlockSpec((AB, NF), lambda a: (a, 0)),
            pl.BlockSpec((AB, NF), lambda a: (a, 0)),
        ],
        out_shape=[
            jax.ShapeDtypeStruct((NP, NF), jnp.float32),
            jax.ShapeDtypeStruct((NP, NF), jnp.float32),
        ],
    )(z2, emb_pad, wn, bn)


def _tc_update_body(r_ref, agg_ref, wu1_ref, bu1_ref, wu2_ref, bu2_ref,
                    wn_ref, bn_ref, rout_ref, rn_ref):
    agg = agg_ref[0] + agg_ref[1]
    t = _ssp(jnp.dot(agg, wu1_ref[...],
                     preferred_element_type=jnp.float32) + bu1_ref[...])
    rnew = r_ref[...] + jnp.dot(t, wu2_ref[...],
                                preferred_element_type=jnp.float32) + bu2_ref[...]
    rn = jnp.dot(rnew, wn_ref[...],
                 preferred_element_type=jnp.float32) + bn_ref[...]
    rout_ref[...] = rnew
    rn_ref[...] = rn


def _tc_update(r, agg2, wu1, bu1, wu2, bu2, wn, bn):
    return pl.pallas_call(
        _tc_update_body,
        grid=(NP // AB,),
        in_specs=[
            pl.BlockSpec((AB, NF), lambda a: (a, 0)),
            pl.BlockSpec((NC, AB, NF), lambda a: (0, a, 0)),
            pl.BlockSpec((NF, NF), lambda a: (0, 0)),
            pl.BlockSpec((1, NF), lambda a: (0, 0)),
            pl.BlockSpec((NF, NF), lambda a: (0, 0)),
            pl.BlockSpec((1, NF), lambda a: (0, 0)),
            pl.BlockSpec((NF, NF), lambda a: (0, 0)),
            pl.BlockSpec((1, NF), lambda a: (0, 0)),
        ],
        out_specs=[
            pl.BlockSpec((AB, NF), lambda a: (a, 0)),
            pl.BlockSpec((AB, NF), lambda a: (a, 0)),
        ],
        out_shape=[
            jax.ShapeDtypeStruct((NP, NF), jnp.float32),
            jax.ShapeDtypeStruct((NP, NF), jnp.float32),
        ],
    )(r, agg2, wu1, bu1, wu2, bu2, wn, bn)


NM_PAD = 512


def _tc_readout_body(r_ref, agg_ref, wu1_ref, bu1_ref, wu2_ref, bu2_ref,
                     wr1_ref, br1_ref, wr2_ref, br2_ref, mol_ref, out_ref):
    agg = agg_ref[0] + agg_ref[1]
    t = _ssp(jnp.dot(agg, wu1_ref[...],
                     preferred_element_type=jnp.float32) + bu1_ref[...])
    rnew = r_ref[...] + jnp.dot(t, wu2_ref[...],
                                preferred_element_type=jnp.float32) + bu2_ref[...]
    t2 = _ssp(jnp.dot(rnew, wr1_ref[...],
                      preferred_element_type=jnp.float32) + br1_ref[...])
    e_at = jnp.dot(t2, wr2_ref[...],
                   preferred_element_type=jnp.float32) + br2_ref[...]
    mids = lax.broadcasted_iota(jnp.int32, (AB, NM_PAD), 1)
    oh = (mol_ref[...] == mids).astype(jnp.float32)
    eng = lax.dot_general(oh, e_at, (((0,), (0,)), ((), ())),
                          preferred_element_type=jnp.float32)

    @pl.when(pl.program_id(0) == 0)
    def _():
        out_ref[...] = jnp.zeros_like(out_ref)

    out_ref[...] += eng


def _tc_readout(r, agg2, wu1, bu1, wu2, bu2, wr1, br1, wr2, br2, mol2):
    return pl.pallas_call(
        _tc_readout_body,
        grid=(NP // AB,),
        in_specs=[
            pl.BlockSpec((AB, NF), lambda a: (a, 0)),
            pl.BlockSpec((NC, AB, NF), lambda a: (0, a, 0)),
            pl.BlockSpec((NF, NF), lambda a: (0, 0)),
            pl.BlockSpec((1, NF), lambda a: (0, 0)),
            pl.BlockSpec((NF, NF), lambda a: (0, 0)),
            pl.BlockSpec((1, NF), lambda a: (0, 0)),
            pl.BlockSpec((NF, NF // 2), lambda a: (0, 0)),
            pl.BlockSpec((1, NF // 2), lambda a: (0, 0)),
            pl.BlockSpec((NF // 2, 1), lambda a: (0, 0)),
            pl.BlockSpec((1, 1), lambda a: (0, 0)),
            pl.BlockSpec((AB, 1), lambda a: (a, 0)),
        ],
        out_specs=pl.BlockSpec((NM_PAD, 1), lambda a: (0, 0)),
        out_shape=jax.ShapeDtypeStruct((NM_PAD, 1), jnp.float32),
    )(r, agg2, wu1, bu1, wu2, bu2, wr1, br1, wr2, br2, mol2)


def kernel(z, xyz, nbr_list, mol_ids, emb, Wef1, bef1, Wef2, bef2,
           Wnf, bnf, Wu1, bu1, Wu2, bu2, Wr1, br1, Wr2, br2):
    a0 = nbr_list[:, 0].astype(jnp.int32)
    a1 = nbr_list[:, 1].astype(jnp.int32)
    a0p = jnp.pad(a0, (0, EP - N_EDGES), constant_values=N_ATOMS)
    a1p = jnp.pad(a1, (0, EP - N_EDGES), constant_values=N_ATOMS)
    a0r = a0p.reshape(ROWS_E, 128)
    a1r = a1p.reshape(ROWS_E, 128)
    a0m = a0p.reshape(EP // KB, KB)
    a1m = a1p.reshape(EP // KB, KB)
    isrc = jnp.concatenate([a0m, a1m], axis=1).reshape(-1)
    idst = jnp.concatenate([a1m, a0m], axis=1).reshape(-1)
    xyzf = xyz.astype(jnp.float32)
    xp = jnp.pad(xyzf[:, 0], (0, NP - N_ATOMS))
    yp = jnp.pad(xyzf[:, 1], (0, NP - N_ATOMS))
    zp = jnp.pad(xyzf[:, 2], (0, NP - N_ATOMS))
    emb_pad = jnp.pad(emb.astype(jnp.float32), ((0, NF - emb.shape[0]), (0, 0)))
    z2 = jnp.pad(z.astype(jnp.int32), (0, NP - N_ATOMS),
                 constant_values=NF - 1).reshape(NP, 1)
    mol2 = jnp.pad(mol_ids.astype(jnp.int32), (0, NP - N_ATOMS),
                   constant_values=N_MOLS).reshape(NP, 1)
    zrows = jnp.zeros((NST, NF), jnp.float32)

    b1 = bef1.astype(jnp.float32).reshape(N_CONV, 1, NG)
    b2 = bef2.astype(jnp.float32).reshape(N_CONV, 1, NF)
    bn = bnf.astype(jnp.float32).reshape(N_CONV, 1, NF)
    b_u1 = bu1.astype(jnp.float32).reshape(N_CONV, 1, NF)
    b_u2 = bu2.astype(jnp.float32).reshape(N_CONV, 1, NF)
    br1_2 = br1.astype(jnp.float32).reshape(1, NF // 2)
    br2_2 = br2.astype(jnp.float32).reshape(1, 1)

    d2 = _sc_d2(xp, yp, zp, a0r, a1r).reshape(EP, 1)

    wedges = [_tc_wedge(d2, Wef1[i], b1[i], Wef2[i], b2[i])
              for i in range(N_CONV)]

    r, rn = _tc_embed(z2, emb_pad, Wnf[0], bn[0])
    for i in range(N_CONV):
        agg2 = _sc_agg(isrc, idst, wedges[i], rn, zrows)
        if i < N_CONV - 1:
            r, rn = _tc_update(r, agg2, Wu1[i], b_u1[i], Wu2[i], b_u2[i],
                               Wnf[i + 1], bn[i + 1])
        else:
            energy = _tc_readout(r, agg2, Wu1[i], b_u1[i], Wu2[i], b_u2[i],
                                 Wr1, br1_2, Wr2, br2_2, mol2)
    return energy[:N_MOLS]


# speedup vs baseline: 1.0428x; 1.0428x over previous
import jax
import jax.numpy as jnp
from jax import lax
from jax.experimental import pallas as pl
from jax.experimental.pallas import tpu as pltpu
from jax.experimental.pallas import tpu_sc as plsc

N_ATOMS = 10000
N_EDGES = 320000
N_MOLS = 500
NF = 128
NG = 32
N_CONV = 3
CUTOFF = 5.0

NC, NS = 2, 16
NW = NC * NS
EP = 327680
NP = 10240
ROWS_E = EP // 128

A_ROWS = EP // 128 // NW


def _sc_d2_body(x_hbm, y_hbm, z_hbm, a0_hbm, a1_hbm, d2_hbm,
                xv, yv, zv, i0v, i1v, d2v):
    c = lax.axis_index("c")
    s = lax.axis_index("s")
    wid = s * NC + c
    pltpu.sync_copy(x_hbm, xv)
    pltpu.sync_copy(y_hbm, yv)
    pltpu.sync_copy(z_hbm, zv)
    base = wid * A_ROWS
    pltpu.sync_copy(a0_hbm.at[pl.ds(base, A_ROWS)], i0v)
    pltpu.sync_copy(a1_hbm.at[pl.ds(base, A_ROWS)], i1v)

    def body(i, _):
        r = i // 8
        co = (i % 8) * 16
        idx0 = i0v[r, pl.ds(co, 16)]
        idx1 = i1v[r, pl.ds(co, 16)]
        dx = plsc.load_gather(xv, [idx1]) - plsc.load_gather(xv, [idx0])
        dy = plsc.load_gather(yv, [idx1]) - plsc.load_gather(yv, [idx0])
        dz = plsc.load_gather(zv, [idx1]) - plsc.load_gather(zv, [idx0])
        d2v[r, pl.ds(co, 16)] = dx * dx + dy * dy + dz * dz + 1e-12
        return 0

    lax.fori_loop(0, A_ROWS * 8, body, 0)
    pltpu.sync_copy(d2v, d2_hbm.at[pl.ds(base, A_ROWS)])


def _sc_d2(xp, yp, zp, a0r, a1r):
    mesh = plsc.VectorSubcoreMesh(core_axis_name="c", subcore_axis_name="s")
    f = pl.kernel(
        _sc_d2_body,
        out_type=jax.ShapeDtypeStruct((ROWS_E, 128), jnp.float32),
        mesh=mesh,
        compiler_params=pltpu.CompilerParams(needs_layout_passes=False),
        scratch_types=[
            pltpu.VMEM((NP,), jnp.float32),
            pltpu.VMEM((NP,), jnp.float32),
            pltpu.VMEM((NP,), jnp.float32),
            pltpu.VMEM((A_ROWS, 128), jnp.int32),
            pltpu.VMEM((A_ROWS, 128), jnp.int32),
            pltpu.VMEM((A_ROWS, 128), jnp.float32),
        ],
    )
    return f(xp, yp, zp, a0r, a1r)


KB = 64
EPC = EP // NC
CHUNKS = EPC // NS // KB
NSH = 10112
NST = NSH // NS


RPC = 2 * KB
BCH = 20
CH0 = 200
CH1 = 120


def _sc_agg_body(isrc_hbm, idst_hbm, w_hbm, rn_hbm, z_hbm, agg_hbm, agg_sh,
                 ibs, ibd, ga, gb, wv, sga, sgb, sw, sca, scb):
    c = lax.axis_index("c")
    s = lax.axis_index("s")
    g = (ga, gb)
    sg = (sga, sgb)
    sc = (sca, scb)
    nch = jnp.where(c == 0, CH0, CH1)
    tchunk = jnp.where(c == 0, s * CH0, NS * CH0 + s * CH1)

    def gfire(lc, b):
        pltpu.async_copy(rn_hbm.at[ibs.at[pl.ds(lc * RPC, RPC)]], g[b], sg[b])

    def gdrain(b):
        pltpu.make_async_copy(
            rn_hbm.at[ibs.at[pl.ds(0, RPC)]], g[b], sg[b]).wait()

    def scfire(lc, b):
        pltpu.async_copy(
            g[b], agg_sh.at[ibd.at[pl.ds(lc * RPC, RPC)]], sc[b], add=True)

    def scdrain(b):
        pltpu.make_async_copy(
            g[b], agg_sh.at[ibd.at[pl.ds(0, RPC)]], sc[b]).wait()

    def wfire(gc):
        pltpu.async_copy(w_hbm.at[pl.ds(gc * KB, KB)], wv, sw)

    def wdrain():
        pltpu.make_async_copy(w_hbm.at[pl.ds(0, KB)], wv, sw).wait()

    pltpu.sync_copy(z_hbm, agg_sh.at[pl.ds(s * NST, NST)])
    plsc.subcore_barrier()

    def batch(m, _):
        @pl.when(m >= 1)
        def _():
            scdrain(1)
        base = tchunk + m * BCH
        pltpu.sync_copy(isrc_hbm.at[pl.ds(base * RPC, BCH * RPC)], ibs)
        pltpu.sync_copy(idst_hbm.at[pl.ds(base * RPC, BCH * RPC)], ibd)
        gfire(0, 0)
        wfire(base)

        def pair(i, _):
            for b in range(2):
                j = 2 * i + b
                nb = 1 - b
                if b == 0:
                    @pl.when(i >= 1)
                    def _():
                        scdrain(nb)
                    gfire(j + 1, nb)
                else:
                    scdrain(nb)

                    @pl.when(i < BCH // 2 - 1)
                    def _():
                        gfire(j + 1, nb)
                gdrain(b)
                wdrain()

                def mul(t, _):
                    for f in range(NF // 16):
                        sl = pl.ds(f * 16, 16)
                        wj = wv[t, sl]
                        g[b][t, sl] = g[b][t, sl] * wj
                        g[b][t + KB, sl] = g[b][t + KB, sl] * wj
                    return 0

                lax.fori_loop(0, KB, mul, 0, unroll=4)
                scfire(j, b)
                @pl.when(j < BCH - 1)
                def _():
                    wfire(base + j + 1)
            return 0

        lax.fori_loop(0, BCH // 2, pair, 0)
        return 0

    lax.fori_loop(0, nch // BCH, batch, 0)
    scdrain(1)
    plsc.subcore_barrier()
    pltpu.sync_copy(agg_sh.at[pl.ds(s * NST, NST)],
                    agg_hbm.at[c, pl.ds(s * NST, NST)])


def _sc_agg(isrc, idst, wedge, rn, zrows):
    mesh = plsc.VectorSubcoreMesh(core_axis_name="c", subcore_axis_name="s")
    f = pl.kernel(
        _sc_agg_body,
        out_type=jax.ShapeDtypeStruct((NC, NP, NF), jnp.float32),
        mesh=mesh,
        compiler_params=pltpu.CompilerParams(needs_layout_passes=False),
        scratch_types=[
            pltpu.VMEM_SHARED((NSH, NF), jnp.float32),
            pltpu.VMEM((BCH * RPC,), jnp.int32),
            pltpu.VMEM((BCH * RPC,), jnp.int32),
            pltpu.VMEM((RPC, NF), jnp.float32),
            pltpu.VMEM((RPC, NF), jnp.float32),
            pltpu.VMEM((KB, NF), jnp.float32),
            pltpu.SemaphoreType.DMA,
            pltpu.SemaphoreType.DMA,
            pltpu.SemaphoreType.DMA,
            pltpu.SemaphoreType.DMA,
            pltpu.SemaphoreType.DMA,
        ],
    )
    return f(isrc, idst, wedge, rn, zrows)


def _ssp(x):
    return jnp.logaddexp(x, 0.0) - jnp.log(2.0)


EB = 2048


def _tc_wedge_body(d2_ref, w1_ref, b1_ref, w2_ref, b2_ref, out_ref):
    width = CUTOFF / (NG - 1)
    coeff = -0.5 / (width * width)
    dist = jnp.sqrt(d2_ref[...])
    offs = lax.broadcasted_iota(jnp.int32, (EB, NG), 1).astype(jnp.float32) * width
    eg = jnp.exp(coeff * (dist - offs) ** 2)
    h = _ssp(jnp.dot(eg, w1_ref[...],
                     preferred_element_type=jnp.float32) + b1_ref[...])
    out_ref[...] = jnp.dot(h, w2_ref[...],
                           preferred_element_type=jnp.float32) + b2_ref[...]


def _tc_wedge(d2, w1, b1, w2, b2):
    return pl.pallas_call(
        _tc_wedge_body,
        grid=(EP // EB,),
        in_specs=[
            pl.BlockSpec((EB, 1), lambda e: (e, 0)),
            pl.BlockSpec((NG, NG), lambda e: (0, 0)),
            pl.BlockSpec((1, NG), lambda e: (0, 0)),
            pl.BlockSpec((NG, NF), lambda e: (0, 0)),
            pl.BlockSpec((1, NF), lambda e: (0, 0)),
        ],
        out_specs=pl.BlockSpec((EB, NF), lambda e: (e, 0)),
        out_shape=jax.ShapeDtypeStruct((EP, NF), jnp.float32),
    )(d2, w1, b1, w2, b2)


AB = 2048


def _tc_embed_body(z_ref, emb_ref, wn_ref, bn_ref, r_ref, rn_ref):
    cols = lax.broadcasted_iota(jnp.int32, (AB, NF), 1)
    oh = (z_ref[...] == cols).astype(jnp.float32)
    r = jnp.dot(oh, emb_ref[...], preferred_element_type=jnp.float32)
    rn = jnp.dot(r, wn_ref[...],
                 preferred_element_type=jnp.float32) + bn_ref[...]
    r_ref[...] = r
    rn_ref[...] = rn


def _tc_embed(z2, emb_pad, wn, bn):
    return pl.pallas_call(
        _tc_embed_body,
        grid=(NP // AB,),
        in_specs=[
            pl.BlockSpec((AB, 1), lambda a: (a, 0)),
            pl.BlockSpec((NF, NF), lambda a: (0, 0)),
            pl.BlockSpec((NF, NF), lambda a: (0, 0)),
            pl.BlockSpec((1, NF), lambda a: (0, 0)),
        ],
        out_specs=[
            pl.BlockSpec((AB, NF), lambda a: (a, 0)),
            pl.BlockSpec((AB, NF), lambda a: (a, 0)),
        ],
        out_shape=[
            jax.ShapeDtypeStruct((NP, NF), jnp.float32),
            jax.ShapeDtypeStruct((NP, NF), jnp.float32),
        ],
    )(z2, emb_pad, wn, bn)


def _tc_update_body(r_ref, agg_ref, wu1_ref, bu1_ref, wu2_ref, bu2_ref,
                    wn_ref, bn_ref, rout_ref, rn_ref):
    agg = agg_ref[0] + agg_ref[1]
    t = _ssp(jnp.dot(agg, wu1_ref[...],
                     preferred_element_type=jnp.float32) + bu1_ref[...])
    rnew = r_ref[...] + jnp.dot(t, wu2_ref[...],
                                preferred_element_type=jnp.float32) + bu2_ref[...]
    rn = jnp.dot(rnew, wn_ref[...],
                 preferred_element_type=jnp.float32) + bn_ref[...]
    rout_ref[...] = rnew
    rn_ref[...] = rn


def _tc_update(r, agg2, wu1, bu1, wu2, bu2, wn, bn):
    return pl.pallas_call(
        _tc_update_body,
        grid=(NP // AB,),
        in_specs=[
            pl.BlockSpec((AB, NF), lambda a: (a, 0)),
            pl.BlockSpec((NC, AB, NF), lambda a: (0, a, 0)),
            pl.BlockSpec((NF, NF), lambda a: (0, 0)),
            pl.BlockSpec((1, NF), lambda a: (0, 0)),
            pl.BlockSpec((NF, NF), lambda a: (0, 0)),
            pl.BlockSpec((1, NF), lambda a: (0, 0)),
            pl.BlockSpec((NF, NF), lambda a: (0, 0)),
            pl.BlockSpec((1, NF), lambda a: (0, 0)),
        ],
        out_specs=[
            pl.BlockSpec((AB, NF), lambda a: (a, 0)),
            pl.BlockSpec((AB, NF), lambda a: (a, 0)),
        ],
        out_shape=[
            jax.ShapeDtypeStruct((NP, NF), jnp.float32),
            jax.ShapeDtypeStruct((NP, NF), jnp.float32),
        ],
    )(r, agg2, wu1, bu1, wu2, bu2, wn, bn)


NM_PAD = 512


def _tc_readout_body(r_ref, agg_ref, wu1_ref, bu1_ref, wu2_ref, bu2_ref,
                     wr1_ref, br1_ref, wr2_ref, br2_ref, mol_ref, out_ref):
    agg = agg_ref[0] + agg_ref[1]
    t = _ssp(jnp.dot(agg, wu1_ref[...],
                     preferred_element_type=jnp.float32) + bu1_ref[...])
    rnew = r_ref[...] + jnp.dot(t, wu2_ref[...],
                                preferred_element_type=jnp.float32) + bu2_ref[...]
    t2 = _ssp(jnp.dot(rnew, wr1_ref[...],
                      preferred_element_type=jnp.float32) + br1_ref[...])
    e_at = jnp.dot(t2, wr2_ref[...],
                   preferred_element_type=jnp.float32) + br2_ref[...]
    mids = lax.broadcasted_iota(jnp.int32, (AB, NM_PAD), 1)
    oh = (mol_ref[...] == mids).astype(jnp.float32)
    eng = lax.dot_general(oh, e_at, (((0,), (0,)), ((), ())),
                          preferred_element_type=jnp.float32)

    @pl.when(pl.program_id(0) == 0)
    def _():
        out_ref[...] = jnp.zeros_like(out_ref)

    out_ref[...] += eng


def _tc_readout(r, agg2, wu1, bu1, wu2, bu2, wr1, br1, wr2, br2, mol2):
    return pl.pallas_call(
        _tc_readout_body,
        grid=(NP // AB,),
        in_specs=[
            pl.BlockSpec((AB, NF), lambda a: (a, 0)),
            pl.BlockSpec((NC, AB, NF), lambda a: (0, a, 0)),
            pl.BlockSpec((NF, NF), lambda a: (0, 0)),
            pl.BlockSpec((1, NF), lambda a: (0, 0)),
            pl.BlockSpec((NF, NF), lambda a: (0, 0)),
            pl.BlockSpec((1, NF), lambda a: (0, 0)),
            pl.BlockSpec((NF, NF // 2), lambda a: (0, 0)),
            pl.BlockSpec((1, NF // 2), lambda a: (0, 0)),
            pl.BlockSpec((NF // 2, 1), lambda a: (0, 0)),
            pl.BlockSpec((1, 1), lambda a: (0, 0)),
            pl.BlockSpec((AB, 1), lambda a: (a, 0)),
        ],
        out_specs=pl.BlockSpec((NM_PAD, 1), lambda a: (0, 0)),
        out_shape=jax.ShapeDtypeStruct((NM_PAD, 1), jnp.float32),
    )(r, agg2, wu1, bu1, wu2, bu2, wr1, br1, wr2, br2, mol2)


def kernel(z, xyz, nbr_list, mol_ids, emb, Wef1, bef1, Wef2, bef2,
           Wnf, bnf, Wu1, bu1, Wu2, bu2, Wr1, br1, Wr2, br2):
    a0 = nbr_list[:, 0].astype(jnp.int32)
    a1 = nbr_list[:, 1].astype(jnp.int32)
    a0p = jnp.pad(a0, (0, EP - N_EDGES), constant_values=N_ATOMS)
    a1p = jnp.pad(a1, (0, EP - N_EDGES), constant_values=N_ATOMS)
    a0r = a0p.reshape(ROWS_E, 128)
    a1r = a1p.reshape(ROWS_E, 128)
    a0m = a0p.reshape(EP // KB, KB)
    a1m = a1p.reshape(EP // KB, KB)
    isrc = jnp.concatenate([a0m, a1m], axis=1).reshape(-1)
    idst = jnp.concatenate([a1m, a0m], axis=1).reshape(-1)
    xyzf = xyz.astype(jnp.float32)
    xp = jnp.pad(xyzf[:, 0], (0, NP - N_ATOMS))
    yp = jnp.pad(xyzf[:, 1], (0, NP - N_ATOMS))
    zp = jnp.pad(xyzf[:, 2], (0, NP - N_ATOMS))
    emb_pad = jnp.pad(emb.astype(jnp.float32), ((0, NF - emb.shape[0]), (0, 0)))
    z2 = jnp.pad(z.astype(jnp.int32), (0, NP - N_ATOMS),
                 constant_values=NF - 1).reshape(NP, 1)
    mol2 = jnp.pad(mol_ids.astype(jnp.int32), (0, NP - N_ATOMS),
                   constant_values=N_MOLS).reshape(NP, 1)
    zrows = jnp.zeros((NST, NF), jnp.float32)

    b1 = bef1.astype(jnp.float32).reshape(N_CONV, 1, NG)
    b2 = bef2.astype(jnp.float32).reshape(N_CONV, 1, NF)
    bn = bnf.astype(jnp.float32).reshape(N_CONV, 1, NF)
    b_u1 = bu1.astype(jnp.float32).reshape(N_CONV, 1, NF)
    b_u2 = bu2.astype(jnp.float32).reshape(N_CONV, 1, NF)
    br1_2 = br1.astype(jnp.float32).reshape(1, NF // 2)
    br2_2 = br2.astype(jnp.float32).reshape(1, 1)

    d2 = _sc_d2(xp, yp, zp, a0r, a1r).reshape(EP, 1)

    wedges = [_tc_wedge(d2, Wef1[i], b1[i], Wef2[i], b2[i])
              for i in range(N_CONV)]

    r, rn = _tc_embed(z2, emb_pad, Wnf[0], bn[0])
    for i in range(N_CONV):
        agg2 = _sc_agg(isrc, idst, wedges[i], rn, zrows)
        if i < N_CONV - 1:
            r, rn = _tc_update(r, agg2, Wu1[i], b_u1[i], Wu2[i], b_u2[i],
                               Wnf[i + 1], bn[i + 1])
        else:
            energy = _tc_readout(r, agg2, Wu1[i], b_u1[i], Wu2[i], b_u2[i],
                                 Wr1, br1_2, Wr2, br2_2, mol2)
    return energy[:N_MOLS]
